# Initial kernel scaffold; baseline (speedup 1.0000x reference)
import jax
import jax.numpy as jnp
from jax.experimental import pallas as pl


def kernel(boxes, im_labels, cls_prob_new, gt_boxes, gt_classes, gt_scores):
    raise NotImplementedError("write your pallas kernel here")


# speedup vs baseline: 1.5731x; 1.5731x over previous
import functools

import jax
import jax.numpy as jnp
from jax import lax
from jax.experimental import pallas as pl
from jax.experimental.pallas import tpu as pltpu
from jax.experimental.pallas import tpu_sc as plsc

N = 20000
G = 64
C = 20
NC = 2
NS = 16
NW = NC * NS
L = 16
PADN = 20480
PER_W = PADN // NW
GROUPS = PER_W // L
PROBW = C + 1

FG_THRESH = 0.5
BG_THRESH = 0.1
EPS = 1e-9
LN2 = 0.6931471805599453
SQRT2 = 1.4142135623730951


def _neg_log(p):
  bits = lax.bitcast_convert_type(p, jnp.int32)
  e = jnp.right_shift(bits, 23) - 127
  mbits = jnp.bitwise_or(jnp.bitwise_and(bits, 0x7FFFFF), 0x3F800000)
  m = lax.bitcast_convert_type(mbits, jnp.float32)
  big = m > SQRT2
  m = jnp.where(big, m * 0.5, m)
  ef = e.astype(jnp.float32) + jnp.where(big, 1.0, 0.0)
  s = (m - 1.0) / (m + 1.0)
  z = s * s
  poly = 1.0 + z * (0.3333333333 + z * (0.2 + z * 0.142857143))
  lnm = 2.0 * s * poly
  return -(ef * LN2 + lnm)


def _sc_body(bx1_h, by1_h, bx2_h, by2_h, prob_h,
             gx1_h, gy1_h, gx2_h, gy2_h, gcls_h, gsc_h,
             out_h,
             bx1_v, by1_v, bx2_v, by2_v, prob_v,
             gx1_v, gy1_v, gx2_v, gy2_v, garea_v,
             gcls_v, gsc_v, acc_v):
  wid = lax.axis_index("s") * NC + lax.axis_index("c")
  base = wid * PER_W

  pltpu.sync_copy(bx1_h.at[pl.ds(base, PER_W)], bx1_v)
  pltpu.sync_copy(by1_h.at[pl.ds(base, PER_W)], by1_v)
  pltpu.sync_copy(bx2_h.at[pl.ds(base, PER_W)], bx2_v)
  pltpu.sync_copy(by2_h.at[pl.ds(base, PER_W)], by2_v)
  pltpu.sync_copy(prob_h.at[pl.ds(base * PROBW, PER_W * PROBW)], prob_v)
  pltpu.sync_copy(gx1_h, gx1_v)
  pltpu.sync_copy(gy1_h, gy1_v)
  pltpu.sync_copy(gx2_h, gx2_v)
  pltpu.sync_copy(gy2_h, gy2_v)
  pltpu.sync_copy(gcls_h, gcls_v)
  pltpu.sync_copy(gsc_h, gsc_v)

  for g in range(G):
    gx2p = gx2_v[g] + 1.0
    gy2p = gy2_v[g] + 1.0
    garea_v[g] = (gx2p - gx1_v[g]) * (gy2p - gy1_v[g])
    gx2_v[g] = gx2p
    gy2_v[g] = gy2p

  iota = lax.iota(jnp.int32, L)

  def group_body(j, acc):
    o = pl.multiple_of(j * L, L)
    x1 = bx1_v[pl.ds(o, L)]
    y1 = by1_v[pl.ds(o, L)]
    x2p = bx2_v[pl.ds(o, L)] + 1.0
    y2p = by2_v[pl.ds(o, L)] + 1.0
    area = (x2p - x1) * (y2p - y1)

    binter = jnp.zeros((L,), jnp.float32)
    bunion = jnp.ones((L,), jnp.float32)
    bestg = jnp.zeros((L,), jnp.int32)
    for g in range(G):
      iw = jnp.maximum(jnp.minimum(x2p, gx2_v[g]) - jnp.maximum(x1, gx1_v[g]),
                       0.0)
      ih = jnp.maximum(jnp.minimum(y2p, gy2_v[g]) - jnp.maximum(y1, gy1_v[g]),
                       0.0)
      inter = iw * ih
      union = area + garea_v[g] - inter
      upd = inter * bunion > binter * union
      binter = jnp.where(upd, inter, binter)
      bunion = jnp.where(upd, union, bunion)
      bestg = jnp.where(upd, g, bestg)

    maxov = binter / bunion
    cls = plsc.load_gather(gcls_v, [bestg])
    wts = plsc.load_gather(gsc_v, [bestg])
    label = jnp.where(maxov < FG_THRESH, 0, cls)
    wts = jnp.where(maxov < BG_THRESH, 0.0, wts)
    lidx = o + iota
    picked = plsc.load_gather(prob_v, [lidx * PROBW + label])
    picked = jnp.maximum(picked, EPS)
    contrib = jnp.where(base + lidx < N, wts * _neg_log(picked), 0.0)
    return acc + contrib

  acc = lax.fori_loop(0, GROUPS, group_body, jnp.zeros((L,), jnp.float32))
  acc_v[...] = acc
  pltpu.sync_copy(acc_v, out_h.at[wid])


@jax.jit
def _dmil_loss(bx1, by1, bx2, by2, prob_flat, gx1, gy1, gx2, gy2, gcls, gsc):
  mesh = plsc.VectorSubcoreMesh(core_axis_name="c", subcore_axis_name="s",
                                num_cores=NC, num_subcores=NS)
  f32 = jnp.float32
  partials = pl.kernel(
      _sc_body,
      out_type=jax.ShapeDtypeStruct((NW, L), f32),
      mesh=mesh,
      compiler_params=pltpu.CompilerParams(needs_layout_passes=False),
      scratch_types=[
          pltpu.VMEM((PER_W,), f32),
          pltpu.VMEM((PER_W,), f32),
          pltpu.VMEM((PER_W,), f32),
          pltpu.VMEM((PER_W,), f32),
          pltpu.VMEM((PER_W * PROBW,), f32),
          pltpu.VMEM((G, L), f32),
          pltpu.VMEM((G, L), f32),
          pltpu.VMEM((G, L), f32),
          pltpu.VMEM((G, L), f32),
          pltpu.VMEM((G, L), f32),
          pltpu.VMEM((G,), jnp.int32),
          pltpu.VMEM((G,), f32),
          pltpu.VMEM((L,), f32),
      ],
  )(bx1, by1, bx2, by2, prob_flat, gx1, gy1, gx2, gy2, gcls, gsc)
  return jnp.sum(partials) / f32(N)


def kernel(boxes, im_labels, cls_prob_new, gt_boxes, gt_classes, gt_scores):
  del im_labels
  pad = PADN - N
  bx1 = jnp.concatenate([boxes[:, 0], jnp.zeros((pad,), jnp.float32)])
  by1 = jnp.concatenate([boxes[:, 1], jnp.zeros((pad,), jnp.float32)])
  bx2 = jnp.concatenate([boxes[:, 2], jnp.zeros((pad,), jnp.float32)])
  by2 = jnp.concatenate([boxes[:, 3], jnp.zeros((pad,), jnp.float32)])
  prob_flat = jnp.concatenate(
      [cls_prob_new, jnp.zeros((pad, PROBW), jnp.float32)]).reshape(-1)
  gx1 = jnp.broadcast_to(gt_boxes[:, 0][:, None], (G, L))
  gy1 = jnp.broadcast_to(gt_boxes[:, 1][:, None], (G, L))
  gx2 = jnp.broadcast_to(gt_boxes[:, 2][:, None], (G, L))
  gy2 = jnp.broadcast_to(gt_boxes[:, 3][:, None], (G, L))
  return _dmil_loss(bx1, by1, bx2, by2, prob_flat, gx1, gy1, gx2, gy2,
                    gt_classes, gt_scores)
